# Initial kernel scaffold; baseline (speedup 1.0000x reference)
import jax
import jax.numpy as jnp
from jax.experimental import pallas as pl


def kernel(input, target):
    raise NotImplementedError("write your pallas kernel here")


# speedup vs baseline: 1.6694x; 1.6694x over previous
import functools

import jax
import jax.numpy as jnp
from jax import lax
from jax.experimental import pallas as pl
from jax.experimental.pallas import tpu as pltpu
from jax.experimental.pallas import tpu_sc as plsc

NC = 2
NS = 16
L = 16
NW = NC * NS

SEG = 800
WINSEGS = 25
TOPK = 64
STEP = 0.001
NEG_INF = float("-inf")
BIG = jnp.int32(1 << 30)


def _lanes():
    return lax.iota(jnp.int32, L)


def _store1(ref, idx, val):
    plsc.store_scatter(ref, [jnp.full((L,), idx, jnp.int32)],
                       jnp.full((L,), val), mask=_lanes() == 0)


def _load1(ref, idx):
    return jnp.max(plsc.load_gather(ref, [jnp.full((L,), idx, jnp.int32)]))


def _argmax_scan(ref, nvregs, unroll):
    lanes = _lanes()

    @pl.loop(0, nvregs, init_carry=(jnp.full((L,), NEG_INF, jnp.float32),
                                    jnp.zeros((L,), jnp.int32)),
             unroll=unroll)
    def scan(j, carry):
        m, mi = carry
        v = ref[pl.ds(j * L, L)]
        gt = v > m
        return (jnp.where(gt, v, m), jnp.where(gt, j * L + lanes, mi))

    m, mi = scan
    best = jnp.max(m)
    cand = jnp.where(m == best, mi, BIG)
    return best, jnp.min(cand)


def _sc_topk_pred(x2d, b):
    nseg = x2d.shape[0] // b
    rpw = b // NW
    nwin = nseg // WINSEGS
    vps = SEG // L
    segpad = ((nseg + L - 1) // L) * L
    mesh = plsc.VectorSubcoreMesh(core_axis_name="c", subcore_axis_name="s",
                                  num_cores=NC, num_subcores=NS)

    @functools.partial(
        pl.kernel,
        out_type=jax.ShapeDtypeStruct((NW * L,), jnp.float32),
        mesh=mesh,
        scratch_types=[
            pltpu.VMEM((WINSEGS, SEG), jnp.float32),
            pltpu.VMEM((WINSEGS, SEG), jnp.float32),
            pltpu.VMEM((segpad,), jnp.float32),
            pltpu.VMEM((TOPK,), jnp.int32),
            pltpu.VMEM((TOPK,), jnp.float32),
            pltpu.VMEM((TOPK, SEG), jnp.float32),
            pltpu.VMEM((L,), jnp.float32),
            pltpu.SemaphoreType.DMA,
            pltpu.SemaphoreType.DMA,
            pltpu.SemaphoreType.DMA,
        ],
        compiler_params=pltpu.CompilerParams(use_tc_tiling_on_sc=False,
                                             needs_layout_passes=False),
    )
    def k(x_hbm, out_hbm, win_a, win_b, segmax, selseg, selmax, gath, outbuf,
          sem_a, sem_b, sem_g):
        wid = lax.axis_index("s") * NC + lax.axis_index("c")
        lanes = _lanes()

        @pl.loop(0, rpw)
        def row_loop(r):
            row = wid * rpw + r
            rowbase = row * nseg

            @pl.loop(0, segpad // L)
            def init_segmax(j):
                segmax[pl.ds(j * L, L)] = jnp.full((L,), NEG_INF, jnp.float32)

            def start(win_ref, sem, w):
                pltpu.make_async_copy(
                    x_hbm.at[pl.ds(rowbase + w * WINSEGS, WINSEGS)],
                    win_ref, sem).start()

            def wait(win_ref, sem):
                pltpu.make_async_copy(
                    x_hbm.at[pl.ds(0, WINSEGS)], win_ref, sem).wait()

            def process(win_ref, w):
                @pl.loop(0, WINSEGS)
                def seg_loop(s):
                    @pl.loop(0, vps,
                             init_carry=jnp.full((L,), NEG_INF, jnp.float32),
                             unroll=10)
                    def acc_loop(j, acc):
                        return jnp.maximum(acc, win_ref[s, pl.ds(j * L, L)])
                    _store1(segmax, w * WINSEGS + s, jnp.max(acc_loop))

            start(win_a, sem_a, 0)

            @pl.loop(0, nwin, step=2)
            def win_loop(t):
                wait(win_a, sem_a)
                start(win_b, sem_b, t + 1)
                process(win_a, t)
                wait(win_b, sem_b)

                @pl.when(t + 2 < nwin)
                def _():
                    start(win_a, sem_a, t + 2)

                process(win_b, t + 1)

            @pl.loop(0, TOPK)
            def sel_loop(ksel):
                best, bi = _argmax_scan(segmax, segpad // L, 8)
                _store1(selseg, ksel, rowbase + bi)
                _store1(selmax, ksel, best)
                _store1(segmax, bi, NEG_INF)

            pltpu.async_copy(x_hbm.at[selseg], gath, sem_g).wait()

            @pl.loop(0, TOPK, init_carry=jnp.zeros((L,), jnp.float32))
            def ext_loop(ke, predv):
                bv, slot = _argmax_scan(selmax, TOPK // L, 4)

                @pl.loop(0, vps, init_carry=jnp.full((L,), BIG, jnp.int32),
                         unroll=10)
                def pos_loop(j, pm):
                    eq = gath[slot, pl.ds(j * L, L)] == bv
                    return jnp.minimum(pm, jnp.where(eq, j * L + lanes, BIG))
                pos = jnp.min(pos_loop)

                jv = pos // L
                vv = gath[slot, pl.ds(jv * L, L)]
                gath[slot, pl.ds(jv * L, L)] = jnp.where(
                    lanes == pos % L, NEG_INF, vv)

                @pl.loop(0, vps, init_carry=jnp.full((L,), NEG_INF, jnp.float32),
                         unroll=10)
                def nm_loop(j, nm):
                    return jnp.maximum(nm, gath[slot, pl.ds(j * L, L)])
                _store1(selmax, slot, jnp.max(nm_loop))

                col = (_load1(selseg, slot) - rowbase) * SEG + pos
                w = col.astype(jnp.float32) * STEP + 1.0
                return predv + bv * w

            @pl.when(r == 0)
            def _():
                outbuf[...] = ext_loop

            @pl.when(r != 0)
            def _():
                outbuf[...] = jnp.where(lanes < L // rpw, outbuf[...], ext_loop)

        pltpu.sync_copy(outbuf, out_hbm.at[pl.ds(wid * L, L)])

    return k(x2d)


def _tc_mse(tpad, ppad):
    def body(t_ref, p_ref, o_ref):
        d = t_ref[...] - p_ref[...]
        o_ref[0, 0] = jnp.sum(d * d) * (1.0 / TOPK)

    return pl.pallas_call(
        body,
        out_shape=jax.ShapeDtypeStruct((1, 1), jnp.float32),
        out_specs=pl.BlockSpec(memory_space=pltpu.SMEM),
    )(tpad, ppad)


def kernel(input, target):
    b, n = input.shape
    nseg = n // SEG
    x2d = input.reshape(b * nseg, SEG)
    raw = _sc_topk_pred(x2d, b)
    rpw = b // NW
    lpr = L // rpw
    preds = raw.reshape(NW, rpw, lpr)[:, :, 0].reshape(b)
    tpad = jnp.zeros((8, 128), jnp.float32).at[:, :8].set(target.reshape(8, 8))
    ppad = jnp.zeros((8, 128), jnp.float32).at[:, :8].set(preds.reshape(8, 8))
    return _tc_mse(tpad, ppad)[0, 0]


# speedup vs baseline: 1.6697x; 1.0002x over previous
import functools

import jax
import jax.numpy as jnp
from jax import lax
from jax.experimental import pallas as pl
from jax.experimental.pallas import tpu as pltpu
from jax.experimental.pallas import tpu_sc as plsc

NC = 2
NS = 16
L = 16
NW = NC * NS

SEG = 800
WINSEGS = 25
TOPK = 64
STEP = 0.001
NEG_INF = float("-inf")
BIG = jnp.int32(1 << 30)


def _lanes():
    return lax.iota(jnp.int32, L)


def _store1(ref, idx, val):
    plsc.store_scatter(ref, [jnp.full((L,), idx, jnp.int32)],
                       jnp.full((L,), val), mask=_lanes() == 0)


def _load1(ref, idx):
    return jnp.max(plsc.load_gather(ref, [jnp.full((L,), idx, jnp.int32)]))


def _argmax_scan(ref, nvregs, unroll):
    lanes = _lanes()

    @pl.loop(0, nvregs, init_carry=(jnp.full((L,), NEG_INF, jnp.float32),
                                    jnp.zeros((L,), jnp.int32)),
             unroll=unroll)
    def scan(j, carry):
        m, mi = carry
        v = ref[pl.ds(j * L, L)]
        gt = v > m
        return (jnp.where(gt, v, m), jnp.where(gt, j * L + lanes, mi))

    m, mi = scan
    best = jnp.max(m)
    cand = jnp.where(m == best, mi, BIG)
    return best, jnp.min(cand)


def _sc_topk_pred(x2d, b):
    nseg = x2d.shape[0] // b
    rpw = b // NW
    nwin = nseg // WINSEGS
    vps = SEG // L
    segpad = ((nseg + L - 1) // L) * L
    mesh = plsc.VectorSubcoreMesh(core_axis_name="c", subcore_axis_name="s",
                                  num_cores=NC, num_subcores=NS)

    @functools.partial(
        pl.kernel,
        out_type=jax.ShapeDtypeStruct((NW * L,), jnp.float32),
        mesh=mesh,
        scratch_types=[
            pltpu.VMEM((WINSEGS, SEG), jnp.float32),
            pltpu.VMEM((WINSEGS, SEG), jnp.float32),
            pltpu.VMEM((segpad,), jnp.float32),
            pltpu.VMEM((TOPK,), jnp.int32),
            pltpu.VMEM((TOPK,), jnp.float32),
            pltpu.VMEM((TOPK, SEG), jnp.float32),
            pltpu.VMEM((L,), jnp.float32),
            pltpu.SemaphoreType.DMA,
            pltpu.SemaphoreType.DMA,
            pltpu.SemaphoreType.DMA,
        ],
        compiler_params=pltpu.CompilerParams(use_tc_tiling_on_sc=False,
                                             needs_layout_passes=False),
    )
    def k(x_hbm, out_hbm, win_a, win_b, segmax, selseg, selmax, gath, outbuf,
          sem_a, sem_b, sem_g):
        wid = lax.axis_index("s") * NC + lax.axis_index("c")
        lanes = _lanes()

        @pl.loop(0, rpw)
        def row_loop(r):
            row = wid * rpw + r
            rowbase = row * nseg

            @pl.loop(0, segpad // L)
            def init_segmax(j):
                segmax[pl.ds(j * L, L)] = jnp.full((L,), NEG_INF, jnp.float32)

            def start(win_ref, sem, w):
                pltpu.make_async_copy(
                    x_hbm.at[pl.ds(rowbase + w * WINSEGS, WINSEGS)],
                    win_ref, sem).start()

            def wait(win_ref, sem):
                pltpu.make_async_copy(
                    x_hbm.at[pl.ds(0, WINSEGS)], win_ref, sem).wait()

            def process(win_ref, w):
                @pl.loop(0, WINSEGS)
                def seg_loop(s):
                    @pl.loop(0, vps,
                             init_carry=jnp.full((L,), NEG_INF, jnp.float32),
                             unroll=10)
                    def acc_loop(j, acc):
                        return jnp.maximum(acc, win_ref[s, pl.ds(j * L, L)])
                    _store1(segmax, w * WINSEGS + s, jnp.max(acc_loop))

            start(win_a, sem_a, 0)

            @pl.loop(0, nwin, step=2)
            def win_loop(t):
                wait(win_a, sem_a)
                start(win_b, sem_b, t + 1)
                process(win_a, t)
                wait(win_b, sem_b)

                @pl.when(t + 2 < nwin)
                def _():
                    start(win_a, sem_a, t + 2)

                process(win_b, t + 1)

            @pl.loop(0, TOPK)
            def sel_loop(ksel):
                best, bi = _argmax_scan(segmax, segpad // L, 8)
                _store1(selseg, ksel, rowbase + bi)
                _store1(selmax, ksel, best)
                _store1(segmax, bi, NEG_INF)

            pltpu.async_copy(x_hbm.at[selseg], gath, sem_g).wait()

            @pl.loop(0, TOPK, init_carry=jnp.zeros((L,), jnp.float32))
            def ext_loop(ke, predv):
                bv, slot = _argmax_scan(selmax, TOPK // L, 4)

                @pl.loop(0, vps, init_carry=jnp.full((L,), BIG, jnp.int32),
                         unroll=10)
                def pos_loop(j, pm):
                    eq = gath[slot, pl.ds(j * L, L)] == bv
                    return jnp.minimum(pm, jnp.where(eq, j * L + lanes, BIG))
                pos = jnp.min(pos_loop)

                jv = pos // L
                vv = gath[slot, pl.ds(jv * L, L)]
                gath[slot, pl.ds(jv * L, L)] = jnp.where(
                    lanes == pos % L, NEG_INF, vv)

                @pl.loop(0, vps, init_carry=jnp.full((L,), NEG_INF, jnp.float32),
                         unroll=10)
                def nm_loop(j, nm):
                    return jnp.maximum(nm, gath[slot, pl.ds(j * L, L)])
                _store1(selmax, slot, jnp.max(nm_loop))

                col = (_load1(selseg, slot) - rowbase) * SEG + pos
                w = col.astype(jnp.float32) * STEP + 1.0
                return predv + bv * w

            @pl.when(r == 0)
            def _():
                outbuf[...] = ext_loop

            @pl.when(r != 0)
            def _():
                outbuf[...] = jnp.where(lanes < L // rpw, outbuf[...], ext_loop)

        pltpu.sync_copy(outbuf, out_hbm.at[pl.ds(wid * L, L)])

    return k(x2d)


def _tc_mse(tdup, pdup):
    def body(t_ref, p_ref, o_ref):
        d = t_ref[...] - p_ref[...]
        o_ref[0, 0] = jnp.sum(d * d) * (1.0 / (TOPK * 8))

    return pl.pallas_call(
        body,
        out_shape=jax.ShapeDtypeStruct((1, 1), jnp.float32),
        out_specs=pl.BlockSpec(memory_space=pltpu.SMEM),
    )(tdup, pdup)


def kernel(input, target):
    b, n = input.shape
    nseg = n // SEG
    x2d = input.reshape(b * nseg, SEG)
    raw = _sc_topk_pred(x2d, b)
    tdup = jnp.repeat(target, 8).reshape(4, 128)
    return _tc_mse(tdup, raw.reshape(4, 128))[0, 0]


# speedup vs baseline: 1.6701x; 1.0003x over previous
import functools

import jax
import jax.numpy as jnp
from jax import lax
from jax.experimental import pallas as pl
from jax.experimental.pallas import tpu as pltpu
from jax.experimental.pallas import tpu_sc as plsc

NC = 2
NS = 16
L = 16
NW = NC * NS

SEG = 800
WINSEGS = 25
WINSZ = WINSEGS * SEG
TOPK = 64
STEP = 0.001
NEG_INF = float("-inf")
BIG = jnp.int32(1 << 30)


def _lanes():
    return lax.iota(jnp.int32, L)


def _store1(ref, idx, val):
    plsc.store_scatter(ref, [jnp.full((L,), idx, jnp.int32)],
                       jnp.full((L,), val), mask=_lanes() == 0)


def _load1(ref, idx):
    return jnp.max(plsc.load_gather(ref, [jnp.full((L,), idx, jnp.int32)]))


def _argmax_scan(ref, nvregs, unroll):
    lanes = _lanes()

    @pl.loop(0, nvregs, init_carry=(jnp.full((L,), NEG_INF, jnp.float32),
                                    jnp.zeros((L,), jnp.int32)),
             unroll=unroll)
    def scan(j, carry):
        m, mi = carry
        v = ref[pl.ds(j * L, L)]
        gt = v > m
        return (jnp.where(gt, v, m), jnp.where(gt, j * L + lanes, mi))

    m, mi = scan
    best = jnp.max(m)
    cand = jnp.where(m == best, mi, BIG)
    return best, jnp.min(cand)


def _sc_topk_pred(x):
    b, n = x.shape
    nseg = n // SEG
    rpw = b // NW
    nwin = nseg // WINSEGS
    vps = SEG // L
    segpad = ((nseg + L - 1) // L) * L
    mesh = plsc.VectorSubcoreMesh(core_axis_name="c", subcore_axis_name="s",
                                  num_cores=NC, num_subcores=NS)

    @functools.partial(
        pl.kernel,
        out_type=jax.ShapeDtypeStruct((NW * L,), jnp.float32),
        mesh=mesh,
        scratch_types=[
            pltpu.VMEM((WINSZ,), jnp.float32),
            pltpu.VMEM((WINSZ,), jnp.float32),
            pltpu.VMEM((segpad,), jnp.float32),
            pltpu.VMEM((TOPK,), jnp.int32),
            pltpu.VMEM((TOPK,), jnp.float32),
            pltpu.VMEM((TOPK, SEG), jnp.float32),
            pltpu.VMEM((L,), jnp.float32),
            pltpu.SemaphoreType.DMA,
            pltpu.SemaphoreType.DMA,
            pltpu.SemaphoreType.DMA,
        ],
        compiler_params=pltpu.CompilerParams(use_tc_tiling_on_sc=False,
                                             needs_layout_passes=False),
    )
    def k(x_hbm, out_hbm, win_a, win_b, segmax, selseg, selmax, gath, outbuf,
          sem_a, sem_b, sem_g):
        wid = lax.axis_index("s") * NC + lax.axis_index("c")
        lanes = _lanes()

        @pl.loop(0, rpw)
        def row_loop(r):
            row = wid * rpw + r

            @pl.loop(0, segpad // L)
            def init_segmax(j):
                segmax[pl.ds(j * L, L)] = jnp.full((L,), NEG_INF, jnp.float32)

            def start(win_ref, sem, w):
                pltpu.make_async_copy(
                    x_hbm.at[row, pl.ds(w * WINSZ, WINSZ)], win_ref, sem
                ).start()

            def wait(win_ref, sem):
                pltpu.make_async_copy(
                    x_hbm.at[0, pl.ds(0, WINSZ)], win_ref, sem).wait()

            def process(win_ref, w):
                @pl.loop(0, WINSEGS)
                def seg_loop(s):
                    @pl.loop(0, vps,
                             init_carry=jnp.full((L,), NEG_INF, jnp.float32),
                             unroll=10)
                    def acc_loop(j, acc):
                        return jnp.maximum(
                            acc, win_ref[pl.ds(s * SEG + j * L, L)])
                    _store1(segmax, w * WINSEGS + s, jnp.max(acc_loop))

            start(win_a, sem_a, 0)

            @pl.loop(0, nwin, step=2)
            def win_loop(t):
                wait(win_a, sem_a)
                start(win_b, sem_b, t + 1)
                process(win_a, t)
                wait(win_b, sem_b)

                @pl.when(t + 2 < nwin)
                def _():
                    start(win_a, sem_a, t + 2)

                process(win_b, t + 1)

            @pl.loop(0, TOPK)
            def sel_loop(ksel):
                best, bi = _argmax_scan(segmax, segpad // L, 8)
                _store1(selseg, ksel, bi)
                _store1(selmax, ksel, best)
                _store1(segmax, bi, NEG_INF)

            @pl.loop(0, TOPK)
            def fetch_loop(kf):
                sid = _load1(selseg, kf)
                off = pl.multiple_of(sid * SEG, 8)
                pltpu.make_async_copy(
                    x_hbm.at[row, pl.ds(off, SEG)], gath.at[kf], sem_g
                ).start()

            pltpu.make_async_copy(
                x_hbm.at[0, pl.ds(0, SEG)], gath, sem_g).wait()

            @pl.loop(0, TOPK, init_carry=jnp.zeros((L,), jnp.float32))
            def ext_loop(ke, predv):
                bv, slot = _argmax_scan(selmax, TOPK // L, 4)

                @pl.loop(0, vps, init_carry=jnp.full((L,), BIG, jnp.int32),
                         unroll=10)
                def pos_loop(j, pm):
                    eq = gath[slot, pl.ds(j * L, L)] == bv
                    return jnp.minimum(pm, jnp.where(eq, j * L + lanes, BIG))
                pos = jnp.min(pos_loop)

                jv = pos // L
                vv = gath[slot, pl.ds(jv * L, L)]
                gath[slot, pl.ds(jv * L, L)] = jnp.where(
                    lanes == pos % L, NEG_INF, vv)

                @pl.loop(0, vps, init_carry=jnp.full((L,), NEG_INF, jnp.float32),
                         unroll=10)
                def nm_loop(j, nm):
                    return jnp.maximum(nm, gath[slot, pl.ds(j * L, L)])
                _store1(selmax, slot, jnp.max(nm_loop))

                col = _load1(selseg, slot) * SEG + pos
                w = col.astype(jnp.float32) * STEP + 1.0
                return predv + bv * w

            @pl.when(r == 0)
            def _():
                outbuf[...] = ext_loop

            @pl.when(r != 0)
            def _():
                outbuf[...] = jnp.where(lanes < L // rpw, outbuf[...], ext_loop)

        pltpu.sync_copy(outbuf, out_hbm.at[pl.ds(wid * L, L)])

    return k(x)


def _tc_mse(tdup, pdup):
    def body(t_ref, p_ref, o_ref):
        d = t_ref[...] - p_ref[...]
        o_ref[0, 0] = jnp.sum(d * d) * (1.0 / (TOPK * 8))

    return pl.pallas_call(
        body,
        out_shape=jax.ShapeDtypeStruct((1, 1), jnp.float32),
        out_specs=pl.BlockSpec(memory_space=pltpu.SMEM),
    )(tdup, pdup)


def kernel(input, target):
    raw = _sc_topk_pred(input)
    tdup = jnp.repeat(target, 8).reshape(4, 128)
    return _tc_mse(tdup, raw.reshape(4, 128))[0, 0]
